# baseline (device time: 56592 ns/iter reference)
import jax
import jax.numpy as jnp
from jax import lax
from jax.experimental import pallas as pl
from jax.experimental.pallas import tpu as pltpu

N_ROWS = 2048
N_COLS = 1024
BUF_ROWS = N_ROWS + 16

_CHUNKS = [128] * 16 + [8] * 17
N_CHUNKS = len(_CHUNKS)


def _chunk_plan(total):
    a = total // 128
    r = total - a * 128
    b = (r + 7) // 8
    c_split = a // 2
    plan = []
    for k in range(16):
        plan.append((k < a, k * 128, k < c_split))
    for j in range(17):
        plan.append((j < b, a * 128 + j * 8, False))
    return plan, c_split


def _a2av(x, perm, meta):
    def body(x_ref, perm_ref, meta_ref, out_ref,
             send_ref, recv_ref, xs_sems, xr_sems, ys_sems, yr_sems):
        my_x = lax.axis_index("x")
        my_y = lax.axis_index("y")
        my_z = lax.axis_index("z")
        x_peer = (1 - my_x, my_y, my_z)
        y_peer = (my_x, 1 - my_y, my_z)

        n_keep = meta_ref[0]
        n_ex = meta_ref[1]
        base_keep = meta_ref[2]
        my_base = meta_ref[3]
        phi = meta_ref[4]
        delta = meta_ref[5]

        i_am_y0 = my_y == 0

        barrier_sem = pltpu.get_barrier_semaphore()
        for nbr in (x_peer, y_peer):
            pl.semaphore_signal(
                barrier_sem, inc=1, device_id=nbr,
                device_id_type=pl.DeviceIdType.MESH,
            )
        pl.semaphore_wait(barrier_sem, 2)

        plan_s, c_s = _chunk_plan(phi + n_ex)
        plan_r, c_r = _chunk_plan(delta + n_ex)

        def x_desc(idx, size, start):
            return pltpu.make_async_remote_copy(
                src_ref=send_ref.at[pl.ds(start, size)],
                dst_ref=recv_ref.at[pl.ds(start, size)],
                send_sem=xs_sems.at[idx],
                recv_sem=xr_sems.at[idx],
                device_id=x_peer,
                device_id_type=pl.DeviceIdType.MESH,
            )

        def y_desc(idx, size, start, h=0):
            return pltpu.make_async_remote_copy(
                src_ref=recv_ref.at[pl.ds(start, size)],
                dst_ref=recv_ref.at[pl.ds(start, size)],
                send_sem=ys_sems.at[idx, h],
                recv_sem=yr_sems.at[idx, h],
                device_id=y_peer,
                device_id_type=pl.DeviceIdType.MESH,
            )

        def y_fwd_start(idx, size, start):
            if size == 128:
                y_desc(idx, 64, start, 0).start()
                y_desc(idx, 64, start + 64, 1).start()
            else:
                y_desc(idx, size, start, 0).start()

        def y_fwd_wait_recv(idx, size, start):
            if size == 128:
                y_desc(idx, 64, start, 0).wait_recv()
                y_desc(idx, 64, start + 64, 1).wait_recv()
            else:
                y_desc(idx, size, start, 0).wait_recv()

        def y_fwd_wait_send(idx, size, start):
            if size == 128:
                y_desc(idx, 64, start, 0).wait_send()
                y_desc(idx, 64, start + 64, 1).wait_send()
            else:
                y_desc(idx, size, start, 0).wait_send()

        split_t = jnp.clip(c_s * 128 - phi, 0, n_ex)
        t_lo = jnp.where(i_am_y0, 0, split_t)
        t_hi = jnp.where(i_am_y0, split_t, n_ex)

        def pack(t, carry):
            send_ref[pl.ds(phi + t, 1), :] = (
                x_ref[pl.ds(perm_ref[t], 1), :]
            )
            return carry

        prev = t_lo
        for k in range(16):
            seg_end = jnp.clip((k + 1) * 128 - phi, t_lo, t_hi)
            lax.fori_loop(prev, seg_end, pack, 0)
            cond, start, in_h0 = plan_s[k]
            mine = cond & (in_h0 == i_am_y0)

            @pl.when(mine)
            def _(idx=k, start=start):
                x_desc(idx, 128, start).start()

            prev = seg_end
        lax.fori_loop(prev, t_hi, pack, 0)
        for j in range(17):
            cond, start, _h = plan_s[16 + j]
            mine = cond & jnp.logical_not(i_am_y0)

            @pl.when(mine)
            def _(idx=16 + j, start=start):
                x_desc(idx, 8, start).start()

        def keep_row(t, carry):
            out_ref[pl.ds(base_keep + t, 1), :] = (
                x_ref[pl.ds(perm_ref[n_ex + t], 1), :]
            )
            return carry

        a_r = (delta + n_ex) // 128
        n_direct = jnp.maximum(jnp.where(i_am_y0, c_r, a_r - c_r), 1)
        kslice = (n_keep + n_direct - 1) // n_direct

        keep_prev = jnp.int32(0)
        for idx, (size, (cond, start, in_h0)) in enumerate(
                zip(_CHUNKS[:16], plan_r[:16])):
            direct = cond & (in_h0 == i_am_y0)

            @pl.when(direct)
            def _(idx=idx, size=size, start=start):
                x_desc(idx, size, start).wait_recv()
                y_fwd_start(idx, size, start)

            keep_end = jnp.where(
                direct, jnp.minimum(keep_prev + kslice, n_keep), keep_prev
            )
            lax.fori_loop(keep_prev, keep_end, keep_row, 0)
            keep_prev = keep_end
        lax.fori_loop(keep_prev, n_keep, keep_row, 0)

        for idx, (size, (cond, start, in_h0)) in enumerate(
                zip(_CHUNKS[16:], plan_r[16:]), start=16):
            direct = cond & (in_h0 == i_am_y0)

            @pl.when(direct)
            def _(idx=idx, size=size, start=start):
                x_desc(idx, size, start).wait_recv()
                y_fwd_start(idx, size, start)

        base_al = pl.multiple_of(my_base - delta, 8)
        total_r = delta + n_ex

        def unpack_row(u, carry):
            out_ref[pl.ds(my_base + (u - delta), 1), :] = (
                recv_ref[pl.ds(u, 1), :]
            )
            return carry

        def unpack_block(q, carry):
            out_ref[pl.ds(base_al + 8 * q, 8), :] = (
                recv_ref[pl.ds(8 * q, 8), :]
            )
            return carry

        def unpack_range(lo, hi):
            lo = jnp.clip(lo, delta, total_r)
            hi = jnp.clip(hi, delta, total_r)
            lo8 = (lo + 7) // 8 * 8
            lax.fori_loop(lo, jnp.minimum(lo8, hi), unpack_row, 0)
            lax.fori_loop(lo8 // 8, hi // 8, unpack_block, 0)
            tail = jnp.maximum(8 * (hi // 8), jnp.minimum(lo8, hi))
            lax.fori_loop(tail, hi, unpack_row, 0)

        split_u = c_r * 128
        unpack_range(
            jnp.where(i_am_y0, delta, split_u),
            jnp.where(i_am_y0, split_u, total_r),
        )

        for idx, (size, (cond, start, in_h0)) in enumerate(zip(_CHUNKS, plan_r)):
            fwd = cond & (in_h0 != i_am_y0)

            @pl.when(fwd)
            def _(idx=idx, size=size, start=start):
                y_fwd_wait_recv(idx, size, start)

        unpack_range(
            jnp.where(i_am_y0, split_u, delta),
            jnp.where(i_am_y0, total_r, split_u),
        )

        for idx, (size, (cond, start, in_h0)) in enumerate(zip(_CHUNKS, plan_s)):
            mine = cond & (in_h0 == i_am_y0)

            @pl.when(mine)
            def _(idx=idx, size=size, start=start):
                x_desc(idx, size, start).wait_send()
        for idx, (size, (cond, start, in_h0)) in enumerate(zip(_CHUNKS, plan_r)):
            direct = cond & (in_h0 == i_am_y0)

            @pl.when(direct)
            def _(idx=idx, size=size, start=start):
                y_fwd_wait_send(idx, size, start)

    return pl.pallas_call(
        body,
        out_shape=jax.ShapeDtypeStruct((N_ROWS, N_COLS), jnp.float32),
        in_specs=[
            pl.BlockSpec(memory_space=pltpu.VMEM),
            pl.BlockSpec(memory_space=pltpu.SMEM),
            pl.BlockSpec(memory_space=pltpu.SMEM),
        ],
        out_specs=pl.BlockSpec(memory_space=pltpu.VMEM),
        scratch_shapes=[
            pltpu.VMEM((BUF_ROWS, N_COLS), jnp.float32),
            pltpu.VMEM((BUF_ROWS, N_COLS), jnp.float32),
            pltpu.SemaphoreType.DMA((N_CHUNKS,)),
            pltpu.SemaphoreType.DMA((N_CHUNKS,)),
            pltpu.SemaphoreType.DMA((N_CHUNKS, 2)),
            pltpu.SemaphoreType.DMA((N_CHUNKS, 2)),
        ],
        compiler_params=pltpu.CompilerParams(collective_id=0),
    )(x, perm, meta)


def kernel(x, dest):
    p = lax.axis_index("x")
    keep = (dest == p).astype(jnp.int32)
    n_keep = jnp.sum(keep)
    n_ex = N_ROWS - n_keep

    perm = jnp.argsort(keep, stable=True).astype(jnp.int32)

    base_keep = jnp.where(p == 0, 0, n_ex)
    my_base = jnp.where(p == 0, n_keep, 0)

    delta = my_base % 8
    phi = jnp.where(p == 0, 0, n_keep % 8)

    meta = jnp.stack(
        [n_keep, n_ex, base_keep, my_base, phi, delta]
    ).astype(jnp.int32)
    return _a2av(x, perm, meta)


# device time: 51343 ns/iter; 1.1022x vs baseline; 1.1022x over previous
import jax
import jax.numpy as jnp
from jax import lax
from jax.experimental import pallas as pl
from jax.experimental.pallas import tpu as pltpu

N_ROWS = 2048
N_COLS = 1024
BUF_ROWS = N_ROWS + 16

_CHUNKS = [128] * 16 + [64, 32, 16, 8]
N_CHUNKS = len(_CHUNKS)


def _chunk_plan(total):
    a = total // 128
    rem8 = (total - a * 128 + 7) // 8
    half_rows = ((total + 15) // 16) * 8
    plan = []
    for k in range(16):
        start = k * 128
        plan.append((k < a, start, start < half_rows))
    s = a * 128
    for unit, size in ((8, 64), (4, 32), (2, 16), (1, 8)):
        present = (rem8 & unit) // unit
        plan.append((present != 0, s, s < half_rows))
        s = s + size * present
    return plan, half_rows


def _a2av(x, perm, meta):
    def body(x_ref, perm_ref, meta_ref, out_ref,
             send_ref, recv_ref, xs_sems, xr_sems, ys_sems, yr_sems):
        my_x = lax.axis_index("x")
        my_y = lax.axis_index("y")
        my_z = lax.axis_index("z")
        x_peer = (1 - my_x, my_y, my_z)
        y_peer = (my_x, 1 - my_y, my_z)

        n_keep = meta_ref[0]
        n_ex = meta_ref[1]
        base_keep = meta_ref[2]
        my_base = meta_ref[3]
        phi = meta_ref[4]
        delta = meta_ref[5]

        i_am_y0 = my_y == 0

        barrier_sem = pltpu.get_barrier_semaphore()
        for nbr in (x_peer, y_peer):
            pl.semaphore_signal(
                barrier_sem, inc=1, device_id=nbr,
                device_id_type=pl.DeviceIdType.MESH,
            )
        pl.semaphore_wait(barrier_sem, 2)

        plan_s, half_s = _chunk_plan(phi + n_ex)
        plan_r, half_r = _chunk_plan(delta + n_ex)

        def x_desc(idx, size, start):
            return pltpu.make_async_remote_copy(
                src_ref=send_ref.at[pl.ds(start, size)],
                dst_ref=recv_ref.at[pl.ds(start, size)],
                send_sem=xs_sems.at[idx],
                recv_sem=xr_sems.at[idx],
                device_id=x_peer,
                device_id_type=pl.DeviceIdType.MESH,
            )

        def y_desc(idx, size, start, h=0):
            return pltpu.make_async_remote_copy(
                src_ref=recv_ref.at[pl.ds(start, size)],
                dst_ref=recv_ref.at[pl.ds(start, size)],
                send_sem=ys_sems.at[idx, h],
                recv_sem=yr_sems.at[idx, h],
                device_id=y_peer,
                device_id_type=pl.DeviceIdType.MESH,
            )

        def y_fwd_start(idx, size, start):
            if size == 128:
                y_desc(idx, 64, start, 0).start()
                y_desc(idx, 64, start + 64, 1).start()
            else:
                y_desc(idx, size, start, 0).start()

        def y_fwd_wait_recv(idx, size, start):
            if size == 128:
                y_desc(idx, 64, start, 0).wait_recv()
                y_desc(idx, 64, start + 64, 1).wait_recv()
            else:
                y_desc(idx, size, start, 0).wait_recv()

        def y_fwd_wait_send(idx, size, start):
            if size == 128:
                y_desc(idx, 64, start, 0).wait_send()
                y_desc(idx, 64, start + 64, 1).wait_send()
            else:
                y_desc(idx, size, start, 0).wait_send()

        a_s = (phi + n_ex) // 128
        bound_s = jnp.minimum(a_s, (half_s + 127) // 128) * 128
        split_t = jnp.clip(bound_s - phi, 0, n_ex)
        t_lo = jnp.where(i_am_y0, 0, split_t)
        t_hi = jnp.where(i_am_y0, split_t, n_ex)

        def pack(t, carry):
            send_ref[pl.ds(phi + t, 1), :] = (
                x_ref[pl.ds(perm_ref[t], 1), :]
            )
            return carry

        prev = t_lo
        for k in range(16):
            seg_end = jnp.clip((k + 1) * 128 - phi, t_lo, t_hi)
            lax.fori_loop(prev, seg_end, pack, 0)
            cond, start, in_h0 = plan_s[k]
            mine = cond & (in_h0 == i_am_y0)

            @pl.when(mine)
            def _(idx=k, start=start):
                x_desc(idx, 128, start).start()

            prev = seg_end
        lax.fori_loop(prev, t_hi, pack, 0)
        for j, size in enumerate(_CHUNKS[16:]):
            cond, start, in_h0 = plan_s[16 + j]
            mine = cond & (in_h0 == i_am_y0)

            @pl.when(mine)
            def _(idx=16 + j, size=size, start=start):
                x_desc(idx, size, start).start()

        def keep_row(t, carry):
            out_ref[pl.ds(base_keep + t, 1), :] = (
                x_ref[pl.ds(perm_ref[n_ex + t], 1), :]
            )
            return carry

        a_r = (delta + n_ex) // 128
        n_dir0 = jnp.minimum(a_r, (half_r + 127) // 128)
        n_direct = jnp.maximum(jnp.where(i_am_y0, n_dir0, a_r - n_dir0), 1)
        kslice = (n_keep + n_direct - 1) // n_direct

        keep_prev = jnp.int32(0)
        for idx, (size, (cond, start, in_h0)) in enumerate(
                zip(_CHUNKS[:16], plan_r[:16])):
            direct = cond & (in_h0 == i_am_y0)

            @pl.when(direct)
            def _(idx=idx, size=size, start=start):
                x_desc(idx, size, start).wait_recv()
                y_fwd_start(idx, size, start)

            keep_end = jnp.where(
                direct, jnp.minimum(keep_prev + kslice, n_keep), keep_prev
            )
            lax.fori_loop(keep_prev, keep_end, keep_row, 0)
            keep_prev = keep_end
        lax.fori_loop(keep_prev, n_keep, keep_row, 0)

        for idx, (size, (cond, start, in_h0)) in enumerate(
                zip(_CHUNKS[16:], plan_r[16:]), start=16):
            direct = cond & (in_h0 == i_am_y0)

            @pl.when(direct)
            def _(idx=idx, size=size, start=start):
                x_desc(idx, size, start).wait_recv()
                y_fwd_start(idx, size, start)

        base_al = pl.multiple_of(my_base - delta, 8)
        total_r = delta + n_ex

        def unpack_row(u, carry):
            out_ref[pl.ds(my_base + (u - delta), 1), :] = (
                recv_ref[pl.ds(u, 1), :]
            )
            return carry

        def unpack_block(q, carry):
            out_ref[pl.ds(base_al + 8 * q, 8), :] = (
                recv_ref[pl.ds(8 * q, 8), :]
            )
            return carry

        def unpack_range(lo, hi):
            lo = jnp.clip(lo, delta, total_r)
            hi = jnp.clip(hi, delta, total_r)
            lo8 = (lo + 7) // 8 * 8
            lax.fori_loop(lo, jnp.minimum(lo8, hi), unpack_row, 0)
            lax.fori_loop(lo8 // 8, hi // 8, unpack_block, 0)
            tail = jnp.maximum(8 * (hi // 8), jnp.minimum(lo8, hi))
            lax.fori_loop(tail, hi, unpack_row, 0)

        split_u = n_dir0 * 128
        unpack_range(
            jnp.where(i_am_y0, delta, split_u),
            jnp.where(i_am_y0, split_u, total_r),
        )

        for idx, (size, (cond, start, in_h0)) in enumerate(zip(_CHUNKS, plan_r)):
            fwd = cond & (in_h0 != i_am_y0)

            @pl.when(fwd)
            def _(idx=idx, size=size, start=start):
                y_fwd_wait_recv(idx, size, start)

        unpack_range(
            jnp.where(i_am_y0, split_u, delta),
            jnp.where(i_am_y0, total_r, split_u),
        )

        for idx, (size, (cond, start, in_h0)) in enumerate(zip(_CHUNKS, plan_s)):
            mine = cond & (in_h0 == i_am_y0)

            @pl.when(mine)
            def _(idx=idx, size=size, start=start):
                x_desc(idx, size, start).wait_send()
        for idx, (size, (cond, start, in_h0)) in enumerate(zip(_CHUNKS, plan_r)):
            direct = cond & (in_h0 == i_am_y0)

            @pl.when(direct)
            def _(idx=idx, size=size, start=start):
                y_fwd_wait_send(idx, size, start)

    return pl.pallas_call(
        body,
        out_shape=jax.ShapeDtypeStruct((N_ROWS, N_COLS), jnp.float32),
        in_specs=[
            pl.BlockSpec(memory_space=pltpu.VMEM),
            pl.BlockSpec(memory_space=pltpu.SMEM),
            pl.BlockSpec(memory_space=pltpu.SMEM),
        ],
        out_specs=pl.BlockSpec(memory_space=pltpu.VMEM),
        scratch_shapes=[
            pltpu.VMEM((BUF_ROWS, N_COLS), jnp.float32),
            pltpu.VMEM((BUF_ROWS, N_COLS), jnp.float32),
            pltpu.SemaphoreType.DMA((N_CHUNKS,)),
            pltpu.SemaphoreType.DMA((N_CHUNKS,)),
            pltpu.SemaphoreType.DMA((N_CHUNKS, 2)),
            pltpu.SemaphoreType.DMA((N_CHUNKS, 2)),
        ],
        compiler_params=pltpu.CompilerParams(collective_id=0),
    )(x, perm, meta)


def kernel(x, dest):
    p = lax.axis_index("x")
    keep = (dest == p).astype(jnp.int32)
    n_keep = jnp.sum(keep)
    n_ex = N_ROWS - n_keep

    perm = jnp.argsort(keep, stable=True).astype(jnp.int32)

    base_keep = jnp.where(p == 0, 0, n_ex)
    my_base = jnp.where(p == 0, n_keep, 0)

    delta = my_base % 8
    phi = jnp.where(p == 0, 0, n_keep % 8)

    meta = jnp.stack(
        [n_keep, n_ex, base_keep, my_base, phi, delta]
    ).astype(jnp.int32)
    return _a2av(x, perm, meta)
